# baseline (speedup 1.0000x reference)
import functools

import jax
import jax.numpy as jnp
from jax import lax
from jax.experimental import pallas as pl
from jax.experimental.pallas import tpu as pltpu
from jax.experimental.pallas import tpu_sc as plsc

NUM_CB = 4
TAB_ROWS = 1030
BT = 8
BATCH = 64
TIME = 2048
OUT_F = NUM_CB * BT
NC = 2
NS = 16
NW = NC * NS
N_UNIT = (BATCH // NW) * NUM_CB
TT = TIME // 128
DSTR = TAB_ROWS + 2


def _body(idx_hbm, tab_hbm, out_hbm, table_s, table_v, idx_v0, idx_v1,
          out_v0, out_v1, out_v2, tab_sem, stage_sem, idx_sem, out_sem):
  idx_bufs = (idx_v0, idx_v1)
  out_bufs = (out_v0, out_v1, out_v2)
  core = lax.axis_index("c")
  sub = lax.axis_index("s")
  wid = sub * NC + core
  b0 = wid * (BATCH // NW)

  def unit_coords(k):
    return b0 + k // NUM_CB, k % NUM_CB

  def start_idx(k):
    b, c = unit_coords(k)
    return pltpu.async_copy(idx_hbm.at[b, c, :], idx_bufs[k % 2], idx_sem)

  def start_out(k):
    b, c = unit_coords(k)
    return pltpu.async_copy(out_bufs[k % 3], out_hbm.at[b, c], out_sem)

  idx_dma = [start_idx(0), start_idx(1)]
  SEC = BT * TAB_ROWS
  sec0_dma = pltpu.async_copy(
      tab_hbm.at[pl.ds(0, SEC)], table_v.at[pl.ds(0, SEC)], tab_sem)
  @pl.when(sub == 0)
  def _():
    pltpu.async_copy(tab_hbm, table_s, stage_sem)
  sec_dma = None

  out_dma = []
  for k in range(N_UNIT):
    idx_dma[k].wait()
    ib = idx_bufs[k % 2]
    if k >= 3:
      out_dma[k - 3].wait()
    ob = out_bufs[k % 3]
    _, c = unit_coords(k)
    if k == 0:
      sec0_dma.wait()
    if k == 1:
      @pl.when(sub == 0)
      def _():
        pltpu.make_async_copy(tab_hbm, table_s, stage_sem).wait()
      plsc.subcore_barrier()
      sec_dma = [
          pltpu.async_copy(table_s.at[pl.ds(cc * SEC, SEC)],
                           table_v.at[pl.ds(cc * SEC, SEC)], tab_sem)
          for cc in range(1, NUM_CB)
      ]
    if 1 <= k <= 3:
      sec_dma[k - 1].wait()

    @plsc.parallel_loop(0, TIME // 16, unroll=8)
    def _(g):
      t0 = pl.multiple_of(g * 16, 16)
      tt = g // 8
      tl0 = pl.multiple_of((g % 8) * 16, 16)
      iv = ib[pl.ds(t0, 16)] + c * (BT * TAB_ROWS)
      for d in range(BT):
        val = plsc.load_gather(table_v, [iv + d * TAB_ROWS])
        ob[tt, d, pl.ds(tl0, 16)] = val

    if k + 2 < N_UNIT:
      idx_dma.append(start_idx(k + 2))
    out_dma.append(start_out(k))
  out_dma[N_UNIT - 3].wait()
  out_dma[N_UNIT - 2].wait()
  out_dma[N_UNIT - 1].wait()


@jax.jit
def _run(index_sequence, tab_fmajor):
  mesh = plsc.VectorSubcoreMesh(core_axis_name="c", subcore_axis_name="s")
  fn = pl.kernel(
      _body,
      out_type=jax.ShapeDtypeStruct((BATCH, NUM_CB, TT, BT, 128), jnp.float32),
      mesh=mesh,
      scratch_types=[
          pltpu.VMEM_SHARED((NUM_CB * BT * TAB_ROWS,), jnp.float32),
          pltpu.VMEM((NUM_CB * BT * TAB_ROWS,), jnp.float32),
          pltpu.VMEM((TIME,), jnp.int32),
          pltpu.VMEM((TIME,), jnp.int32),
          pltpu.VMEM((TT, BT, 128), jnp.float32),
          pltpu.VMEM((TT, BT, 128), jnp.float32),
          pltpu.VMEM((TT, BT, 128), jnp.float32),
          pltpu.SemaphoreType.DMA,
          pltpu.SemaphoreType.DMA,
          pltpu.SemaphoreType.DMA,
          pltpu.SemaphoreType.DMA,
      ],
      compiler_params=pltpu.CompilerParams(needs_layout_passes=False),
  )
  return fn(index_sequence, tab_fmajor)


def kernel(index_sequence, speaker_embedding, tables, is_inference):
  del speaker_embedding, is_inference
  tab_fmajor = jnp.transpose(tables, (0, 2, 1)).reshape(-1)
  out = _run(index_sequence, tab_fmajor)
  return jnp.transpose(out, (0, 2, 4, 1, 3)).reshape(BATCH, TIME, OUT_F)


# speedup vs baseline: 1.0432x; 1.0432x over previous
import functools

import jax
import jax.numpy as jnp
from jax import lax
from jax.experimental import pallas as pl
from jax.experimental.pallas import tpu as pltpu
from jax.experimental.pallas import tpu_sc as plsc

NUM_CB = 4
TAB_ROWS = 1030
BT = 8
BATCH = 64
TIME = 2048
OUT_F = NUM_CB * BT
NC = 2
NS = 16
NW = NC * NS
N_UNIT = (BATCH // NW) * NUM_CB
TT = TIME // 128
DSTR = TAB_ROWS + 2


def _body(idx_hbm, tab_hbm, out_hbm, table_s, table_v, idx_v0, idx_v1,
          out_v0, out_v1, out_v2, tab_sem, idx_sem, out_sem):
  idx_bufs = (idx_v0, idx_v1)
  out_bufs = (out_v0, out_v1, out_v2)
  core = lax.axis_index("c")
  sub = lax.axis_index("s")
  wid = sub * NC + core
  b0 = wid * (BATCH // NW)

  def unit_coords(k):
    return b0 + k // NUM_CB, k % NUM_CB

  def start_idx(k):
    b, c = unit_coords(k)
    return pltpu.async_copy(idx_hbm.at[b, c, :], idx_bufs[k % 2], idx_sem)

  def start_out(k):
    b, c = unit_coords(k)
    return pltpu.async_copy(out_bufs[k % 3], out_hbm.at[b, c], out_sem)

  idx_dma = [start_idx(0), start_idx(1)]

  @pl.when(sub == 0)
  def _():
    pltpu.async_copy(tab_hbm, table_s, tab_sem).wait()
  plsc.subcore_barrier()
  SEC = BT * TAB_ROWS
  pltpu.sync_copy(table_s.at[pl.ds(0, SEC)], table_v.at[pl.ds(0, SEC)])
  sec_dma = [
      pltpu.async_copy(table_s.at[pl.ds(c * SEC, SEC)],
                       table_v.at[pl.ds(c * SEC, SEC)], tab_sem)
      for c in range(1, NUM_CB)
  ]

  out_dma = []
  for k in range(N_UNIT):
    idx_dma[k].wait()
    ib = idx_bufs[k % 2]
    if k >= 3:
      out_dma[k - 3].wait()
    ob = out_bufs[k % 3]
    _, c = unit_coords(k)
    if 1 <= k <= 3:
      sec_dma[k - 1].wait()

    @plsc.parallel_loop(0, TIME // 16, unroll=8)
    def _(g):
      t0 = pl.multiple_of(g * 16, 16)
      tt = g // 8
      tl0 = pl.multiple_of((g % 8) * 16, 16)
      iv = ib[pl.ds(t0, 16)] + c * (BT * TAB_ROWS)
      for d in range(BT):
        val = plsc.load_gather(table_v, [iv + d * TAB_ROWS])
        ob[tt, d, pl.ds(tl0, 16)] = val

    if k + 2 < N_UNIT:
      idx_dma.append(start_idx(k + 2))
    out_dma.append(start_out(k))
  out_dma[N_UNIT - 3].wait()
  out_dma[N_UNIT - 2].wait()
  out_dma[N_UNIT - 1].wait()


@jax.jit
def _run(index_sequence, tab_fmajor):
  mesh = plsc.VectorSubcoreMesh(core_axis_name="c", subcore_axis_name="s")
  fn = pl.kernel(
      _body,
      out_type=jax.ShapeDtypeStruct((BATCH, NUM_CB, TT, BT, 128), jnp.float32),
      mesh=mesh,
      scratch_types=[
          pltpu.VMEM_SHARED((NUM_CB * BT * TAB_ROWS,), jnp.float32),
          pltpu.VMEM((NUM_CB * BT * TAB_ROWS,), jnp.float32),
          pltpu.VMEM((TIME,), jnp.int32),
          pltpu.VMEM((TIME,), jnp.int32),
          pltpu.VMEM((TT, BT, 128), jnp.float32),
          pltpu.VMEM((TT, BT, 128), jnp.float32),
          pltpu.VMEM((TT, BT, 128), jnp.float32),
          pltpu.SemaphoreType.DMA,
          pltpu.SemaphoreType.DMA,
          pltpu.SemaphoreType.DMA,
      ],
      compiler_params=pltpu.CompilerParams(needs_layout_passes=False),
  )
  return fn(index_sequence, tab_fmajor)


def kernel(index_sequence, speaker_embedding, tables, is_inference):
  del speaker_embedding, is_inference
  tab_fmajor = jnp.transpose(tables, (0, 2, 1)).reshape(-1)
  out = _run(index_sequence, tab_fmajor)
  return jnp.transpose(out, (0, 2, 4, 1, 3)).reshape(BATCH, TIME, OUT_F)
